# baseline (speedup 1.0000x reference)
import jax
import jax.numpy as jnp
from jax import lax
from jax.experimental import pallas as pl
from jax.experimental.pallas import tpu as pltpu

UNROLL = 16
NACC = 4


def _cox_body(t_s, e_s, t2, p2, ind2, out_ref):
    B = t_s.shape[1]
    trows = t2[...]
    acc0 = tuple(jnp.zeros_like(trows) for _ in range(NACC))

    z = jnp.float32(0.0)

    def body(it, accs):
        j = it * UNROLL
        accs = list(accs)
        for g in range(UNROLL // 3):
            jj = j + 3 * g
            t0, e0 = t_s[0, jj], e_s[0, jj]
            t1, e1 = t_s[0, jj + 1], e_s[0, jj + 1]
            t2, e2 = t_s[0, jj + 2], e_s[0, jj + 2]
            sw = t0 < t1
            ta = jnp.where(sw, t1, t0)
            ea = jnp.where(sw, e1, e0)
            tb = jnp.where(sw, t0, t1)
            eb = jnp.where(sw, e0, e1)
            sw = ta < t2
            thi = jnp.where(sw, t2, ta)
            ehi = jnp.where(sw, e2, ea)
            tc_ = jnp.where(sw, ta, t2)
            ec_ = jnp.where(sw, ea, e2)
            sw = tc_ < tb
            tmid = jnp.where(sw, tb, tc_)
            emid = jnp.where(sw, eb, ec_)
            tlo = jnp.where(sw, tc_, tb)
            elo = jnp.where(sw, ec_, eb)
            c1 = ehi
            c2 = ehi + emid
            c3 = c2 + elo
            a = g % NACC
            v = jnp.where(thi >= trows, c1, z)
            v = jnp.where(tmid >= trows, c2, v)
            v = jnp.where(tlo >= trows, c3, v)
            accs[a] = accs[a] + v
        for u in range(3 * (UNROLL // 3), UNROLL):
            tj = t_s[0, j + u]
            ej = e_s[0, j + u]
            a = u % NACC
            accs[a] = accs[a] + jnp.where(tj >= trows, ej, z)
        return tuple(accs)

    accs = lax.fori_loop(0, B // UNROLL, body, acc0)
    s = (accs[0] + accs[1]) + (accs[2] + accs[3])
    ind = ind2[...].astype(jnp.float32)
    diffs = p2[...] - jnp.log(s)
    num = jnp.sum(ind * diffs)
    den = jnp.sum(ind)
    out_ref[...] = (-(num / den)).reshape(1, 1)


@jax.jit
def kernel(pred, gt_indicator, gt_time):
    B = pred.shape[0]
    t_s = gt_time.reshape(1, B)
    e_s = jnp.exp(pred.reshape(1, B))
    t2 = gt_time.reshape(32, 128)
    p2 = pred.reshape(32, 128)
    ind2 = gt_indicator.reshape(32, 128)

    out = pl.pallas_call(
        _cox_body,
        in_specs=[
            pl.BlockSpec(memory_space=pltpu.SMEM),
            pl.BlockSpec(memory_space=pltpu.SMEM),
            pl.BlockSpec((32, 128), lambda: (0, 0)),
            pl.BlockSpec((32, 128), lambda: (0, 0)),
            pl.BlockSpec((32, 128), lambda: (0, 0)),
        ],
        out_specs=pl.BlockSpec((1, 1), lambda: (0, 0)),
        out_shape=jax.ShapeDtypeStruct((1, 1), jnp.float32),
    )(t_s, e_s, t2, p2, ind2)

    return out.reshape(())


# speedup vs baseline: 1.3122x; 1.3122x over previous
import jax
import jax.numpy as jnp
from jax import lax
from jax.experimental import pallas as pl
from jax.experimental.pallas import tpu as pltpu

UNROLL = 16
NACC = 4


def _cox_body(t_s, e_s, t2, p2, ind2, out_ref):
    B = t_s.shape[1]
    trows = t2[...]
    acc0 = tuple(jnp.zeros_like(trows) for _ in range(NACC))

    def body(it, accs):
        j = it * UNROLL
        accs = list(accs)
        for u in range(UNROLL):
            tj = t_s[0, j + u]
            ej = e_s[0, j + u]
            a = u % NACC
            accs[a] = accs[a] + jnp.where(tj >= trows, ej,
                                          jnp.float32(0.0))
        return tuple(accs)

    accs = lax.fori_loop(0, B // UNROLL, body, acc0)
    s = (accs[0] + accs[1]) + (accs[2] + accs[3])
    ind = ind2[...].astype(jnp.float32)
    diffs = p2[...] - jnp.log(s)
    num = jnp.sum(ind * diffs)
    den = jnp.sum(ind)
    out_ref[...] = (-(num / den)).reshape(1, 1)


@jax.jit
def kernel(pred, gt_indicator, gt_time):
    B = pred.shape[0]
    t_s = gt_time.reshape(1, B)
    e_s = jnp.exp(pred.reshape(1, B))
    t2 = gt_time.reshape(32, 128)
    p2 = pred.reshape(32, 128)
    ind2 = gt_indicator.reshape(32, 128)

    out = pl.pallas_call(
        _cox_body,
        in_specs=[
            pl.BlockSpec(memory_space=pltpu.SMEM),
            pl.BlockSpec(memory_space=pltpu.SMEM),
            pl.BlockSpec((32, 128), lambda: (0, 0)),
            pl.BlockSpec((32, 128), lambda: (0, 0)),
            pl.BlockSpec((32, 128), lambda: (0, 0)),
        ],
        out_specs=pl.BlockSpec((1, 1), lambda: (0, 0)),
        out_shape=jax.ShapeDtypeStruct((1, 1), jnp.float32),
    )(t_s, e_s, t2, p2, ind2)

    return out.reshape(())
